# Initial kernel scaffold; baseline (speedup 1.0000x reference)
import jax
import jax.numpy as jnp
from jax.experimental import pallas as pl


def kernel(output, price_f):
    raise NotImplementedError("write your pallas kernel here")


# speedup vs baseline: 76.4993x; 76.4993x over previous
import functools

import jax
import jax.numpy as jnp
from jax.experimental import pallas as pl
from jax.experimental.pallas import tpu as pltpu

_GAMMA = 0.99
_ATOMS = 51
_PSIZE = 4


def _circ(x, r):
    if r == 0:
        return x
    return jnp.concatenate([x[..., r:], x[..., :r]], axis=-1)


def _tile_kernel(o_ref, tgt_ref, acc_ref, *, tt, t_total):
    pb = pl.program_id(0)
    pt = pl.program_id(1)

    o = o_ref[0].reshape(tt * _PSIZE, 8, _ATOMS)
    tgt = tgt_ref[0].reshape(tt * _PSIZE, 8, 1)

    j = jax.lax.broadcasted_iota(jnp.int32, (1, 1, _ATOMS), 2).astype(jnp.float32)

    pe = jnp.exp(o)
    inv = 1.0 / jnp.sum(pe, axis=-1, keepdims=True)
    pd = pe * inv
    lp = jnp.log(pd + 1e-8)
    pv = jnp.sum((0.04 * j - 1.0) * pd, axis=-1, keepdims=True)

    gap = tgt - pv
    c = (gap + 0.01) * 25.0

    bu = c + _GAMMA * j
    b = jnp.clip(bu, 0.0, 50.0)
    lf = jnp.maximum(jnp.ceil(b), 1.0) - 1.0
    f = b - lf
    d = lf - j

    mid = (bu > 0.0) & (bu < 50.0)
    dm = jnp.where(mid, d, 1e9)
    s = jnp.min(dm, axis=-1, keepdims=True)
    sm = jnp.where(s < 1e8, s, 0.0).astype(jnp.int32) % _ATOMS

    lps = lp
    for k in range(6):
        r = 1 << k
        cond = ((sm >> k) & 1) == 1
        lps = jnp.where(cond, _circ(lps, r), lps)
    lps1 = _circ(lps, 1)
    lps2 = _circ(lps1, 1)

    g_mid = jnp.where(d == s,
                      (1.0 - f) * lps + f * lps1,
                      (1.0 - f) * lps1 + f * lps2)
    lp0 = lp[..., 0:1]
    lp50 = lp[..., _ATOMS - 1:_ATOMS]
    g = jnp.where(bu <= 0.0, lp0, jnp.where(bu >= 50.0, lp50, g_mid))

    row = jnp.sum(pd * g, axis=-1, keepdims=True)

    r0 = jax.lax.broadcasted_iota(jnp.int32, (tt * _PSIZE, 1, 1), 0)
    t_idx = pt * tt + r0 // _PSIZE
    row = jnp.where(t_idx >= _PSIZE, row, 0.0)

    partial = jnp.sum(row, axis=0, keepdims=True)
    partial = jnp.sum(partial, axis=1, keepdims=True).reshape(1, 1)

    @pl.when((pb == 0) & (pt == 0))
    def _init():
        acc_ref[...] = partial

    @pl.when((pb != 0) | (pt != 0))
    def _acc():
        acc_ref[...] = acc_ref[...] + partial


def _pick_tt(t):
    for cand in range(128, 0, -1):
        if t % cand == 0:
            return cand
    return 1


@jax.jit
def kernel(output, price_f):
    bsz, t, p, dsz, atoms = output.shape
    assert atoms == _ATOMS and p == _PSIZE

    pf = price_f[:, :, None, :]
    parts = []
    for i in range(_PSIZE):
        s, e = i + 1, -(_PSIZE - i - 1)
        parts.append(pf[:, s:] if e == 0 else pf[:, s:e])
    target = jnp.concatenate(parts, axis=2)[..., None]

    tt = _pick_tt(t)
    nt = t // tt

    acc = pl.pallas_call(
        functools.partial(_tile_kernel, tt=tt, t_total=t),
        grid=(bsz, nt),
        in_specs=[
            pl.BlockSpec((1, tt, p, dsz, atoms), lambda b, tb: (b, tb, 0, 0, 0)),
            pl.BlockSpec((1, tt, p, dsz, 1), lambda b, tb: (b, tb, 0, 0, 0)),
        ],
        out_specs=pl.BlockSpec((1, 1), lambda b, tb: (0, 0)),
        out_shape=jax.ShapeDtypeStruct((1, 1), jnp.float32),
        compiler_params=pltpu.CompilerParams(
            dimension_semantics=("arbitrary", "arbitrary"),
        ),
    )(output, target)

    n = bsz * (t - _PSIZE) * p * dsz
    return -acc[0, 0] / n


# speedup vs baseline: 76.7871x; 1.0038x over previous
import functools

import jax
import jax.numpy as jnp
from jax.experimental import pallas as pl
from jax.experimental.pallas import tpu as pltpu

_GAMMA = 0.99
_ATOMS = 51
_PSIZE = 4


def _circ(x, r):
    if r == 0:
        return x
    return jnp.concatenate([x[..., r:], x[..., :r]], axis=-1)


def _tile_kernel(o_ref, tgt_ref, acc_ref, *, tt, t_total):
    pb = pl.program_id(0)
    pt = pl.program_id(1)

    o = o_ref[0].reshape(tt * _PSIZE, 8, _ATOMS)
    tgt = tgt_ref[0].reshape(tt * _PSIZE, 8, 1)

    j = jax.lax.broadcasted_iota(jnp.int32, (1, 1, _ATOMS), 2).astype(jnp.float32)

    pe = jnp.exp(o)
    inv = 1.0 / jnp.sum(pe, axis=-1, keepdims=True)
    pd = pe * inv
    lp = jnp.log(pd + 1e-8)
    pv = jnp.sum((0.04 * j - 1.0) * pd, axis=-1, keepdims=True)

    gap = tgt - pv
    c = (gap + 0.01) * 25.0

    bu = c + _GAMMA * j
    b = jnp.clip(bu, 0.0, 50.0)
    lf = jnp.maximum(jnp.ceil(b), 1.0) - 1.0
    f = b - lf
    d = lf - j

    mid = (bu > 0.0) & (bu < 50.0)
    dm = jnp.where(mid, d, 1e9)
    s = jnp.min(dm, axis=-1, keepdims=True)
    sm = jnp.where(s < 1e8, s, 0.0).astype(jnp.int32) % _ATOMS

    lps = lp
    for k in range(6):
        r = 1 << k
        cond = ((sm >> k) & 1) == 1
        lps = jnp.where(cond, _circ(lps, r), lps)
    lps1 = _circ(lps, 1)
    lps2 = _circ(lps1, 1)

    g_mid = jnp.where(d == s,
                      (1.0 - f) * lps + f * lps1,
                      (1.0 - f) * lps1 + f * lps2)
    lp0 = lp[..., 0:1]
    lp50 = lp[..., _ATOMS - 1:_ATOMS]
    g = jnp.where(bu <= 0.0, lp0, jnp.where(bu >= 50.0, lp50, g_mid))

    row = jnp.sum(pd * g, axis=-1, keepdims=True)

    r0 = jax.lax.broadcasted_iota(jnp.int32, (tt * _PSIZE, 1, 1), 0)
    t_idx = pt * tt + r0 // _PSIZE
    row = jnp.where(t_idx >= _PSIZE, row, 0.0)

    partial = jnp.sum(row, axis=0, keepdims=True)
    partial = jnp.sum(partial, axis=1, keepdims=True).reshape(1, 1, 1, 1)
    acc_ref[...] = partial


def _pick_tt(t):
    for cand in range(128, 0, -1):
        if t % cand == 0:
            return cand
    return 1


@jax.jit
def kernel(output, price_f):
    bsz, t, p, dsz, atoms = output.shape
    assert atoms == _ATOMS and p == _PSIZE

    pf = price_f[:, :, None, :]
    parts = []
    for i in range(_PSIZE):
        s, e = i + 1, -(_PSIZE - i - 1)
        parts.append(pf[:, s:] if e == 0 else pf[:, s:e])
    target = jnp.concatenate(parts, axis=2)[..., None]

    tt = _pick_tt(t)
    nt = t // tt

    acc = pl.pallas_call(
        functools.partial(_tile_kernel, tt=tt, t_total=t),
        grid=(bsz, nt),
        in_specs=[
            pl.BlockSpec((1, tt, p, dsz, atoms), lambda b, tb: (b, tb, 0, 0, 0)),
            pl.BlockSpec((1, tt, p, dsz, 1), lambda b, tb: (b, tb, 0, 0, 0)),
        ],
        out_specs=pl.BlockSpec((1, 1, 1, 1), lambda b, tb: (b, tb, 0, 0)),
        out_shape=jax.ShapeDtypeStruct((bsz, nt, 1, 1), jnp.float32),
        compiler_params=pltpu.CompilerParams(
            dimension_semantics=("parallel", "parallel"),
        ),
    )(output, target)

    n = bsz * (t - _PSIZE) * p * dsz
    return -jnp.sum(acc) / n


# speedup vs baseline: 161.3451x; 2.1012x over previous
import functools

import jax
import jax.numpy as jnp
from jax.experimental import pallas as pl
from jax.experimental.pallas import tpu as pltpu

_GAMMA = 0.99
_ATOMS = 51
_PSIZE = 4


def _circ(x, r):
    if r == 0:
        return x
    return jnp.concatenate([x[..., r:], x[..., :r]], axis=-1)


def _tile_kernel(o_ref, tgt_ref, acc_ref, *, tt, t_total):
    pb = pl.program_id(0)
    pt = pl.program_id(1)

    o = o_ref[0].reshape(tt * _PSIZE, 8, _ATOMS)
    tgt = tgt_ref[0].reshape(tt * _PSIZE, 8, 1)

    j = jax.lax.broadcasted_iota(jnp.int32, (1, 1, _ATOMS), 2).astype(jnp.float32)

    pe = jnp.exp(o)
    inv = 1.0 / jnp.sum(pe, axis=-1, keepdims=True)
    pd = pe * inv
    lp = jnp.log(pd + 1e-8)
    pv = jnp.sum((0.04 * j - 1.0) * pd, axis=-1, keepdims=True)

    gap = tgt - pv
    c = (gap + 0.01) * 25.0

    b = jnp.clip(c + _GAMMA * j, 0.0, 50.0)
    lf = jnp.maximum(jnp.ceil(b), 1.0) - 1.0
    f = b - lf
    li = lf.astype(jnp.int32)
    g_l = jnp.take_along_axis(lp, li, axis=-1)
    g_u = jnp.take_along_axis(lp, li + 1, axis=-1)
    g = (1.0 - f) * g_l + f * g_u

    row = jnp.sum(pd * g, axis=-1, keepdims=True)

    r0 = jax.lax.broadcasted_iota(jnp.int32, (tt * _PSIZE, 1, 1), 0)
    t_idx = pt * tt + r0 // _PSIZE
    row = jnp.where(t_idx >= _PSIZE, row, 0.0)

    partial = jnp.sum(row, axis=0, keepdims=True)
    partial = jnp.sum(partial, axis=1, keepdims=True).reshape(1, 1, 1, 1)
    acc_ref[...] = partial


def _pick_tt(t):
    for cand in range(128, 0, -1):
        if t % cand == 0:
            return cand
    return 1


@jax.jit
def kernel(output, price_f):
    bsz, t, p, dsz, atoms = output.shape
    assert atoms == _ATOMS and p == _PSIZE

    pf = price_f[:, :, None, :]
    parts = []
    for i in range(_PSIZE):
        s, e = i + 1, -(_PSIZE - i - 1)
        parts.append(pf[:, s:] if e == 0 else pf[:, s:e])
    target = jnp.concatenate(parts, axis=2)[..., None]

    tt = _pick_tt(t)
    nt = t // tt

    acc = pl.pallas_call(
        functools.partial(_tile_kernel, tt=tt, t_total=t),
        grid=(bsz, nt),
        in_specs=[
            pl.BlockSpec((1, tt, p, dsz, atoms), lambda b, tb: (b, tb, 0, 0, 0)),
            pl.BlockSpec((1, tt, p, dsz, 1), lambda b, tb: (b, tb, 0, 0, 0)),
        ],
        out_specs=pl.BlockSpec((1, 1, 1, 1), lambda b, tb: (b, tb, 0, 0)),
        out_shape=jax.ShapeDtypeStruct((bsz, nt, 1, 1), jnp.float32),
        compiler_params=pltpu.CompilerParams(
            dimension_semantics=("parallel", "parallel"),
        ),
    )(output, target)

    n = bsz * (t - _PSIZE) * p * dsz
    return -jnp.sum(acc) / n
